# baseline (speedup 1.0000x reference)
import functools

import jax
import jax.numpy as jnp
from jax import lax
from jax.experimental import pallas as pl
from jax.experimental.pallas import tpu as pltpu
from jax.experimental.pallas import tpu_sc as plsc

_BATCH = 16384
_EMBED_DIM = 64

_info = plsc.get_sparse_core_info()
_NC, _NS = _info.num_cores, _info.num_subcores
_NW = _NC * _NS
_B_PER_W = _BATCH // _NW

_mesh = plsc.VectorSubcoreMesh(core_axis_name="c", subcore_axis_name="s")


@functools.partial(
    pl.kernel,
    mesh=_mesh,
    out_type=jax.ShapeDtypeStruct((_BATCH, _EMBED_DIM), jnp.float32),
    scratch_types=[
        pltpu.VMEM((_B_PER_W,), jnp.int32),
        pltpu.VMEM((_B_PER_W, _EMBED_DIM), jnp.float32),
        pltpu.SemaphoreType.DMA,
    ],
    compiler_params=pltpu.CompilerParams(
        use_tc_tiling_on_sc=False,
        disable_bounds_checks=True,
        disable_semaphore_checks=True,
        skip_device_barrier=True,
    ),
)
def _gather_kernel(ids_hbm, table_hbm, out_hbm, idx_v, rows_v, sem):
    wid = lax.axis_index("s") * _NC + lax.axis_index("c")
    base = wid * _B_PER_W
    pltpu.sync_copy(ids_hbm.at[pl.ds(base, _B_PER_W)], idx_v)
    pltpu.async_copy(table_hbm.at[idx_v], rows_v, sem).wait()
    pltpu.sync_copy(rows_v, out_hbm.at[pl.ds(base, _B_PER_W)])


def kernel(ids, table):
    return _gather_kernel(ids.astype(jnp.int32), table)


# speedup vs baseline: 1.2218x; 1.2218x over previous
import functools

import jax
import jax.numpy as jnp
from jax import lax
from jax.experimental import pallas as pl
from jax.experimental.pallas import tpu as pltpu
from jax.experimental.pallas import tpu_sc as plsc

_BATCH = 16384
_EMBED_DIM = 64
_ROWS = 1001

_info = plsc.get_sparse_core_info()
_NC, _NS, _L = _info.num_cores, _info.num_subcores, _info.num_lanes
_NW = _NC * _NS

_D_BLOCKS = _EMBED_DIM // 8
_B_SPLIT = _NW // _D_BLOCKS
_B_PER_W = _BATCH // _B_SPLIT
_JTILES = _B_PER_W // 128
_TBLK = 8 * _ROWS

_mesh = plsc.VectorSubcoreMesh(core_axis_name="c", subcore_axis_name="s")


@functools.partial(
    pl.kernel,
    mesh=_mesh,
    out_type=jax.ShapeDtypeStruct((_EMBED_DIM * _BATCH,), jnp.float32),
    scratch_types=[
        pltpu.VMEM((_B_PER_W,), jnp.int32),
        pltpu.VMEM((_TBLK,), jnp.float32),
        pltpu.VMEM((8 * _B_PER_W,), jnp.float32),
    ],
    compiler_params=pltpu.CompilerParams(needs_layout_passes=False),
)
def _gather_kernel(ids_hbm, ttab_hbm, out_hbm, idx_v, tbl_v, blk_v):
    wid = lax.axis_index("s") * _NC + lax.axis_index("c")
    d_blk = wid // _B_SPLIT
    b_q = wid % _B_SPLIT
    pltpu.sync_copy(ids_hbm.at[pl.ds(b_q * _B_PER_W, _B_PER_W)], idx_v)
    pltpu.sync_copy(ttab_hbm.at[pl.ds(d_blk * _TBLK, _TBLK)], tbl_v)

    def body(j, carry):
        for k in range(8):
            ids16 = idx_v[pl.ds(j * 128 + k * _L, _L)]
            for r in range(8):
                blk_v[pl.ds(j * 1024 + r * 128 + k * _L, _L)] = (
                    plsc.load_gather(tbl_v, [ids16 + (r * _ROWS)])
                )
        return carry

    lax.fori_loop(0, _JTILES, body, 0)
    pltpu.sync_copy(
        blk_v,
        out_hbm.at[pl.ds((d_blk * 128 + b_q * _JTILES) * 1024, 8 * _B_PER_W)],
    )


def kernel(ids, table):
    tt1d = jnp.reshape(table.T, (-1,))
    out1d = _gather_kernel(ids.astype(jnp.int32), tt1d)
    out4 = jnp.reshape(out1d, (8, 128, 8, 128))
    return jnp.reshape(jnp.transpose(out4, (1, 3, 0, 2)), (_BATCH, _EMBED_DIM))


# speedup vs baseline: 1.6842x; 1.3785x over previous
import functools

import jax
import jax.numpy as jnp
from jax import lax
from jax.experimental import pallas as pl
from jax.experimental.pallas import tpu as pltpu
from jax.experimental.pallas import tpu_sc as plsc

_BATCH = 16384
_EMBED_DIM = 64
_ROWS = 1001

_info = plsc.get_sparse_core_info()
_NC, _NS, _L = _info.num_cores, _info.num_subcores, _info.num_lanes
_NW = _NC * _NS

_D_BLOCKS = _EMBED_DIM // 8
_B_SPLIT = _NW // _D_BLOCKS
_B_PER_W = _BATCH // _B_SPLIT
_JTILES = _B_PER_W // 128
_TBLK = 8 * _ROWS

_mesh = plsc.VectorSubcoreMesh(core_axis_name="c", subcore_axis_name="s")


@functools.partial(
    pl.kernel,
    mesh=_mesh,
    out_type=jax.ShapeDtypeStruct((_EMBED_DIM * _BATCH,), jnp.float32),
    scratch_types=[
        pltpu.VMEM((_B_PER_W,), jnp.int32),
        pltpu.VMEM((_TBLK,), jnp.float32),
        pltpu.VMEM((8 * _B_PER_W,), jnp.float32),
        pltpu.SemaphoreType.DMA,
    ],
    compiler_params=pltpu.CompilerParams(
        needs_layout_passes=False,
        disable_bounds_checks=True,
        disable_semaphore_checks=True,
    ),
)
def _gather_kernel(ids_hbm, ttab_hbm, out_hbm, idx_v, tbl_v, blk_v, osem):
    wid = lax.axis_index("s") * _NC + lax.axis_index("c")
    d_blk = wid // _B_SPLIT
    b_q = wid % _B_SPLIT
    in1 = pltpu.async_copy(
        ids_hbm.at[pl.ds(b_q * _B_PER_W, _B_PER_W)], idx_v, osem
    )
    in2 = pltpu.async_copy(
        ttab_hbm.at[pl.ds(d_blk * _TBLK, _TBLK)], tbl_v, osem
    )
    in1.wait()
    in2.wait()

    out_base = (d_blk * 128 + b_q * _JTILES) * 1024
    copies = []
    for c in range(2):
        @plsc.parallel_loop(c * 128, (c + 1) * 128, unroll=4)
        def body(jk):
            base = (jk // 8) * 1024 + (jk % 8) * _L
            ids16 = idx_v[pl.ds(jk * _L, _L)]
            for r in range(8):
                blk_v[pl.ds(base + r * 128, _L)] = (
                    plsc.load_gather(tbl_v, [ids16 + (r * _ROWS)])
                )

        copies.append(
            pltpu.async_copy(
                blk_v.at[pl.ds(c * 16384, 16384)],
                out_hbm.at[pl.ds(out_base + c * 16384, 16384)],
                osem,
            )
        )
    for cp in copies:
        cp.wait()


def kernel(ids, table):
    tt1d = jnp.reshape(table.T, (-1,))
    out1d = _gather_kernel(ids.astype(jnp.int32), tt1d)
    out4 = jnp.reshape(out1d, (8, 128, 8, 128))
    return jnp.reshape(jnp.transpose(out4, (1, 3, 0, 2)), (_BATCH, _EMBED_DIM))


# speedup vs baseline: 1.6866x; 1.0014x over previous
import functools

import jax
import jax.numpy as jnp
from jax import lax
from jax.experimental import pallas as pl
from jax.experimental.pallas import tpu as pltpu
from jax.experimental.pallas import tpu_sc as plsc

_BATCH = 16384
_EMBED_DIM = 64
_ROWS = 1001

_info = plsc.get_sparse_core_info()
_NC, _NS, _L = _info.num_cores, _info.num_subcores, _info.num_lanes
_NW = _NC * _NS

_D_BLOCKS = _EMBED_DIM // 8
_B_SPLIT = _NW // _D_BLOCKS
_B_PER_W = _BATCH // _B_SPLIT
_JTILES = _B_PER_W // 128
_TBLK = 8 * _ROWS

_mesh = plsc.VectorSubcoreMesh(core_axis_name="c", subcore_axis_name="s")


@functools.partial(
    pl.kernel,
    mesh=_mesh,
    out_type=jax.ShapeDtypeStruct((_EMBED_DIM * _BATCH,), jnp.float32),
    scratch_types=[
        pltpu.VMEM((_B_PER_W,), jnp.int32),
        pltpu.VMEM((_TBLK,), jnp.float32),
        pltpu.VMEM((8 * _B_PER_W,), jnp.float32),
        pltpu.SemaphoreType.DMA,
    ],
    compiler_params=pltpu.CompilerParams(
        needs_layout_passes=False,
        allow_input_fusion=[False, True],
        disable_bounds_checks=True,
        disable_semaphore_checks=True,
    ),
)
def _gather_kernel(ids_hbm, ttab_hbm, out_hbm, idx_v, tbl_v, blk_v, osem):
    wid = lax.axis_index("s") * _NC + lax.axis_index("c")
    d_blk = wid // _B_SPLIT
    b_q = wid % _B_SPLIT
    in1 = pltpu.async_copy(
        ids_hbm.at[pl.ds(b_q * _B_PER_W, _B_PER_W)], idx_v, osem
    )
    in2 = pltpu.async_copy(
        ttab_hbm.at[pl.ds(d_blk * _TBLK, _TBLK)], tbl_v, osem
    )
    in1.wait()
    in2.wait()

    out_base = (d_blk * 128 + b_q * _JTILES) * 1024
    copies = []
    for c in range(2):
        @plsc.parallel_loop(c * 128, (c + 1) * 128, unroll=4)
        def body(jk):
            base = (jk // 8) * 1024 + (jk % 8) * _L
            ids16 = idx_v[pl.ds(jk * _L, _L)]
            for r in range(8):
                blk_v[pl.ds(base + r * 128, _L)] = (
                    plsc.load_gather(tbl_v, [ids16 + (r * _ROWS)])
                )

        copies.append(
            pltpu.async_copy(
                blk_v.at[pl.ds(c * 16384, 16384)],
                out_hbm.at[pl.ds(out_base + c * 16384, 16384)],
                osem,
            )
        )
    for cp in copies:
        cp.wait()


def kernel(ids, table):
    tt1d = jnp.reshape(table.T, (-1,))
    out1d = _gather_kernel(ids.astype(jnp.int32), tt1d)
    out4 = jnp.reshape(out1d, (8, 128, 8, 128))
    return jnp.reshape(jnp.transpose(out4, (1, 3, 0, 2)), (_BATCH, _EMBED_DIM))
